# Initial kernel scaffold; baseline (speedup 1.0000x reference)
import jax
import jax.numpy as jnp
from jax.experimental import pallas as pl


def kernel(input, weight):
    raise NotImplementedError("write your pallas kernel here")


# speedup vs baseline: 1.5475x; 1.5475x over previous
import functools

import jax
import jax.numpy as jnp
from jax import lax
from jax.experimental import pallas as pl
from jax.experimental.pallas import tpu as pltpu
from jax.experimental.pallas import tpu_sc as plsc

_NUM_WORKERS = 32
_CHUNK = 1024


@functools.partial(jax.jit, static_argnames=("n_per_w", "n_chunks", "dim"))
def _emb_call(flat_idx, weight, *, n_per_w, n_chunks, dim):
    n_total = flat_idx.shape[0]
    mesh = plsc.VectorSubcoreMesh(core_axis_name="c", subcore_axis_name="s")

    @functools.partial(
        pl.kernel,
        mesh=mesh,
        out_type=jax.ShapeDtypeStruct((n_total, dim), jnp.float32),
        scratch_types=[
            pltpu.VMEM((_CHUNK,), jnp.int32),
            pltpu.VMEM((_CHUNK, dim), jnp.float32),
            pltpu.SemaphoreType.DMA,
        ],
        compiler_params=pltpu.CompilerParams(use_tc_tiling_on_sc=False),
    )
    def emb(idx_hbm, table_hbm, out_hbm, idx_v, rows_v, sem):
        wid = lax.axis_index("s") * 2 + lax.axis_index("c")
        base = wid * n_per_w

        def body(g, carry):
            off = base + g * _CHUNK
            pltpu.sync_copy(idx_hbm.at[pl.ds(off, _CHUNK)], idx_v)
            pltpu.async_copy(table_hbm.at[idx_v], rows_v, sem).wait()
            pltpu.sync_copy(rows_v, out_hbm.at[pl.ds(off, _CHUNK)])
            return carry

        lax.fori_loop(0, n_chunks, body, 0)

    return emb(flat_idx, weight)


def kernel(input, weight):
    b, f = input.shape
    _, dim = weight.shape
    n_total = b * f
    n_per_w = n_total // _NUM_WORKERS
    n_chunks = n_per_w // _CHUNK
    flat_idx = input.reshape(n_total).astype(jnp.int32)
    out = _emb_call(flat_idx, weight, n_per_w=n_per_w, n_chunks=n_chunks, dim=dim)
    return out.reshape(b, f, dim)


# speedup vs baseline: 1.5727x; 1.0163x over previous
import functools

import jax
import jax.numpy as jnp
from jax import lax
from jax.experimental import pallas as pl
from jax.experimental.pallas import tpu as pltpu
from jax.experimental.pallas import tpu_sc as plsc

_NUM_WORKERS = 32
_CHUNK = 1024


@functools.partial(jax.jit, static_argnames=("n_per_w", "n_chunks", "dim"))
def _emb_call(flat_idx, weight, *, n_per_w, n_chunks, dim):
    n_total = flat_idx.shape[0]
    mesh = plsc.VectorSubcoreMesh(core_axis_name="c", subcore_axis_name="s")

    @functools.partial(
        pl.kernel,
        mesh=mesh,
        out_type=jax.ShapeDtypeStruct((n_total, dim), jnp.float32),
        scratch_types=[
            pltpu.VMEM((n_per_w,), jnp.int32),
            pltpu.VMEM((2, _CHUNK, dim), jnp.float32),
            pltpu.SemaphoreType.DMA,
            pltpu.SemaphoreType.DMA,
        ],
        compiler_params=pltpu.CompilerParams(use_tc_tiling_on_sc=False),
    )
    def emb(idx_hbm, table_hbm, out_hbm, idx_v, rows_v, gsem, ssem):
        wid = lax.axis_index("s") * 2 + lax.axis_index("c")
        base = wid * n_per_w
        pltpu.sync_copy(idx_hbm.at[pl.ds(base, n_per_w)], idx_v)

        gathers = [None] * n_chunks
        stores = [None] * n_chunks
        gathers[0] = pltpu.async_copy(
            table_hbm.at[idx_v.at[pl.ds(0, _CHUNK)]], rows_v.at[0], gsem)
        for g in range(n_chunks):
            b = g % 2
            gathers[g].wait()
            stores[g] = pltpu.async_copy(
                rows_v.at[b], out_hbm.at[pl.ds(base + g * _CHUNK, _CHUNK)], ssem)
            if g + 1 < n_chunks:
                if g >= 1:
                    stores[g - 1].wait()
                gathers[g + 1] = pltpu.async_copy(
                    table_hbm.at[idx_v.at[pl.ds((g + 1) * _CHUNK, _CHUNK)]],
                    rows_v.at[b ^ 1], gsem)
        if n_chunks >= 2:
            stores[n_chunks - 2].wait()
        stores[n_chunks - 1].wait()

    return emb(flat_idx, weight)


def kernel(input, weight):
    b, f = input.shape
    _, dim = weight.shape
    n_total = b * f
    n_per_w = n_total // _NUM_WORKERS
    n_chunks = n_per_w // _CHUNK
    flat_idx = input.reshape(n_total).astype(jnp.int32)
    out = _emb_call(flat_idx, weight, n_per_w=n_per_w, n_chunks=n_chunks, dim=dim)
    return out.reshape(b, f, dim)
